# baseline (speedup 1.0000x reference)
import functools

import jax
import jax.numpy as jnp
from jax import lax
from jax.experimental import pallas as pl
from jax.experimental.pallas import tpu as pltpu
from jax.experimental.pallas import tpu_sc as plsc

B, L, V, C = 32, 8192, 16, 4
LANES = 16
NC = 2
NCH = 4
CHUNK = L // NCH


def _one_hot4_sc(input_ids, id2chan):
    mesh = plsc.VectorSubcoreMesh(core_axis_name="c", subcore_axis_name="s")

    @functools.partial(
        pl.kernel,
        mesh=mesh,
        out_type=jax.ShapeDtypeStruct((B, C, L), jnp.float32),
        scratch_types=[
            pltpu.VMEM((L,), jnp.int32),
            pltpu.VMEM((C, L), jnp.float32),
            pltpu.VMEM((V,), jnp.int32),
            pltpu.VMEM((2 * LANES,), jnp.int32),
            pltpu.SemaphoreType.DMA,
            pltpu.SemaphoreType.DMA,
            pltpu.SemaphoreType.DMA,
        ],
    )
    def k(ids_hbm, tab_hbm, out_hbm, ids_v, out_v, tab_v, red_v, sem, tsem, isem):
        wid = lax.axis_index("s") * NC + lax.axis_index("c")
        tab_cp = pltpu.async_copy(tab_hbm, tab_v, tsem)
        for j in range(NCH):
            pltpu.async_copy(
                ids_hbm.at[wid, pl.ds(j * CHUNK, CHUNK)],
                ids_v.at[pl.ds(j * CHUNK, CHUNK)],
                isem,
            )
        tab_cp.wait()

        tab = tab_v[...]
        bit = jnp.left_shift(jnp.int32(1), lax.iota(jnp.int32, LANES))
        zero = jnp.zeros((LANES,), jnp.int32)

        def allreduce_or(x):
            for sh in (1, 2, 4, 8):
                red_v[pl.ds(0, LANES)] = x
                red_v[pl.ds(LANES, LANES)] = x
                x = jnp.bitwise_or(x, red_v[pl.ds(sh, LANES)])
            return x

        packed = []
        for c in (0, 2):
            lo = jnp.where(tab == c, bit, zero)
            hi = jnp.where(tab == c + 1, jnp.left_shift(bit, 16), zero)
            packed.append(allreduce_or(jnp.bitwise_or(lo, hi)))
        halfmask = jnp.full((LANES,), 0xFFFF, jnp.int32)
        mvec = [
            jnp.bitwise_and(packed[0], halfmask),
            jnp.bitwise_and(jnp.right_shift(packed[0], 16), halfmask),
            jnp.bitwise_and(packed[1], halfmask),
            jnp.bitwise_and(jnp.right_shift(packed[1], 16), halfmask),
        ]

        def chunk_body(j, carry):
            base = j * CHUNK
            pltpu.make_async_copy(
                ids_hbm.at[wid, pl.ds(0, CHUNK)],
                ids_v.at[pl.ds(0, CHUNK)],
                isem,
            ).wait()

            @plsc.parallel_loop(0, CHUNK, step=LANES, unroll=16)
            def body(i):
                ids = ids_v[pl.ds(base + i, LANES)]
                for c in range(C):
                    hit = jnp.bitwise_and(jnp.right_shift(mvec[c], ids), 1)
                    out_v[c, pl.ds(base + i, LANES)] = hit.astype(jnp.float32)

            pltpu.async_copy(
                out_v.at[:, pl.ds(base, CHUNK)],
                out_hbm.at[wid, :, pl.ds(base, CHUNK)],
                sem,
            )
            return carry

        lax.fori_loop(0, NCH, chunk_body, 0)
        for _ in range(NCH):
            pltpu.make_async_copy(
                out_v.at[:, pl.ds(0, CHUNK)],
                out_hbm.at[wid, :, pl.ds(0, CHUNK)],
                sem,
            ).wait()

    return k(input_ids, id2chan)


def kernel(input_ids, id2chan):
    return _one_hot4_sc(input_ids, id2chan.astype(jnp.int32))


# speedup vs baseline: 1.0345x; 1.0345x over previous
import functools

import jax
import jax.numpy as jnp
from jax import lax
from jax.experimental import pallas as pl
from jax.experimental.pallas import tpu as pltpu
from jax.experimental.pallas import tpu_sc as plsc

B, L, V, C = 32, 8192, 16, 4
LANES = 16
NC = 2
NCH = 4
CHUNK = L // NCH


def _one_hot4_sc(input_ids, id2chan):
    mesh = plsc.VectorSubcoreMesh(core_axis_name="c", subcore_axis_name="s")

    @functools.partial(
        pl.kernel,
        mesh=mesh,
        out_type=jax.ShapeDtypeStruct((B, C, L), jnp.float32),
        scratch_types=[
            pltpu.VMEM((L,), jnp.int32),
            pltpu.VMEM((C, L), jnp.float32),
            pltpu.VMEM((V,), jnp.int32),
            pltpu.VMEM((2 * LANES,), jnp.int32),
            pltpu.SemaphoreType.DMA,
            pltpu.SemaphoreType.DMA,
            pltpu.SemaphoreType.DMA,
        ],
    )
    def k(ids_hbm, tab_hbm, out_hbm, ids_v, out_v, tab_v, red_v, sem, tsem, isem):
        wid = lax.axis_index("s") * NC + lax.axis_index("c")
        tab_cp = pltpu.async_copy(tab_hbm, tab_v, tsem)
        ids_cp = pltpu.async_copy(ids_hbm.at[wid], ids_v, isem)
        tab_cp.wait()

        tab = tab_v[...]
        bit = jnp.left_shift(jnp.int32(1), lax.iota(jnp.int32, LANES))
        zero = jnp.zeros((LANES,), jnp.int32)

        def allreduce_or(x):
            for sh in (1, 2, 4, 8):
                red_v[pl.ds(0, LANES)] = x
                red_v[pl.ds(LANES, LANES)] = x
                x = jnp.bitwise_or(x, red_v[pl.ds(sh, LANES)])
            return x

        packed = []
        for c in (0, 2):
            lo = jnp.where(tab == c, bit, zero)
            hi = jnp.where(tab == c + 1, jnp.left_shift(bit, 16), zero)
            packed.append(allreduce_or(jnp.bitwise_or(lo, hi)))
        halfmask = jnp.full((LANES,), 0xFFFF, jnp.int32)
        mvec = [
            jnp.bitwise_and(packed[0], halfmask),
            jnp.bitwise_and(jnp.right_shift(packed[0], 16), halfmask),
            jnp.bitwise_and(packed[1], halfmask),
            jnp.bitwise_and(jnp.right_shift(packed[1], 16), halfmask),
        ]

        ids_cp.wait()

        def chunk_body(j, carry):
            base = j * CHUNK

            @plsc.parallel_loop(0, CHUNK, step=LANES, unroll=16)
            def body(i):
                ids = ids_v[pl.ds(base + i, LANES)]
                for c in range(C):
                    hit = jnp.bitwise_and(jnp.right_shift(mvec[c], ids), 1)
                    out_v[c, pl.ds(base + i, LANES)] = hit.astype(jnp.float32)

            pltpu.async_copy(
                out_v.at[:, pl.ds(base, CHUNK)],
                out_hbm.at[wid, :, pl.ds(base, CHUNK)],
                sem,
            )
            return carry

        lax.fori_loop(0, NCH, chunk_body, 0)
        for _ in range(NCH):
            pltpu.make_async_copy(
                out_v.at[:, pl.ds(0, CHUNK)],
                out_hbm.at[wid, :, pl.ds(0, CHUNK)],
                sem,
            ).wait()

    return k(input_ids, id2chan)


def kernel(input_ids, id2chan):
    return _one_hot4_sc(input_ids, id2chan.astype(jnp.int32))
